# baseline (speedup 1.0000x reference)
import functools

import jax
import jax.numpy as jnp
from jax import lax
from jax.experimental import pallas as pl
from jax.experimental.pallas import tpu as pltpu
from jax.experimental.pallas import tpu_sc as plsc

_NC = 2
_NS = 16
_NW = _NC * _NS

_IB = 128
_PW = 128
_L = 16


@functools.lru_cache(maxsize=None)
def _build(hist, batch, vocab, d):
  mesh = plsc.VectorSubcoreMesh(
      core_axis_name="c", subcore_axis_name="s",
      num_cores=_NC, num_subcores=_NS)

  @functools.partial(
      pl.kernel,
      out_type=jax.ShapeDtypeStruct((hist, d, batch), jnp.float32),
      mesh=mesh,
      scratch_types=[
          pltpu.VMEM((2, _IB), jnp.int32),
          pltpu.VMEM((2, _IB, _PW), jnp.float32),
          pltpu.VMEM((2, d, _IB), jnp.float32),
          pltpu.SemaphoreType.DMA,
      ],
      compiler_params=pltpu.CompilerParams(needs_layout_passes=False),
  )
  def lookup(idx_hbm, table_hbm, out_hbm, idx_v, rows_v, outt_v, gsem):
    wid = lax.axis_index("s") * _NC + lax.axis_index("c")
    b0 = wid * _IB

    def fire(h, s):
      pltpu.sync_copy(idx_hbm.at[h, pl.ds(b0, _IB)], idx_v.at[s])
      pltpu.make_async_copy(
          table_hbm.at[idx_v.at[s]], rows_v.at[s], gsem).start()

    lanes = lax.iota(jnp.int32, _L)

    def retire(h, s):
      pltpu.make_async_copy(
          table_hbm.at[idx_v.at[s]], rows_v.at[s], gsem).wait()

      def col(c, carry):
        csplat = jnp.full((_L,), c, jnp.int32)
        for k in range(_IB // _L):
          v = plsc.load_gather(rows_v.at[s], [lanes + (k * _L), csplat])
          outt_v[s, c, pl.ds(k * _L, _L)] = v
        return carry

      lax.fori_loop(0, d, col, 0)
      pltpu.sync_copy(outt_v.at[s], out_hbm.at[h, :, pl.ds(b0, _IB)])

    fire(0, 0)
    fire(1, 1)

    def body(i, carry):
      for b in range(2):
        h = i * 2 + b
        retire(h, b)

        @pl.when(h + 2 < hist)
        def _():
          fire(h + 2, b)
      return carry

    lax.fori_loop(0, hist // 2, body, 0)

  return lookup


def kernel(inputs, embeddings):
  b, h = inputs.shape
  vocab, d = embeddings.shape
  idx_t = jnp.transpose(inputs.astype(jnp.int32))
  tpad = jnp.pad(embeddings, ((0, 0), (0, _PW - d)))
  out = _build(h, b, vocab, d)(idx_t, tpad)
  return jnp.transpose(out, (2, 0, 1))


# speedup vs baseline: 1.5560x; 1.5560x over previous
import functools

import jax
import jax.numpy as jnp
from jax import lax
from jax.experimental import pallas as pl
from jax.experimental.pallas import tpu as pltpu
from jax.experimental.pallas import tpu_sc as plsc

_NC = 2
_NS = 16
_NW = _NC * _NS

_IB = 128
_PW = 128
_L = 16
_NBUF = 4


@functools.lru_cache(maxsize=None)
def _build(hist, batch, vocab, d):
  mesh = plsc.VectorSubcoreMesh(
      core_axis_name="c", subcore_axis_name="s",
      num_cores=_NC, num_subcores=_NS)

  @functools.partial(
      pl.kernel,
      out_type=jax.ShapeDtypeStruct((hist, d, batch), jnp.float32),
      mesh=mesh,
      scratch_types=[
          pltpu.VMEM((hist, _IB), jnp.int32),
          pltpu.VMEM((_NBUF, _IB, _PW), jnp.float32),
          pltpu.VMEM((2, d, _IB), jnp.float32),
          pltpu.SemaphoreType.DMA,
          pltpu.SemaphoreType.DMA,
      ],
      compiler_params=pltpu.CompilerParams(needs_layout_passes=False),
  )
  def lookup(idx_hbm, table_hbm, out_hbm, idx_v, rows_v, outt_v, gsem, osem):
    wid = lax.axis_index("s") * _NC + lax.axis_index("c")
    b0 = wid * _IB

    pltpu.sync_copy(idx_hbm.at[:, pl.ds(b0, _IB)], idx_v)

    def fire(h, j):
      pltpu.make_async_copy(
          table_hbm.at[idx_v.at[h]], rows_v.at[j], gsem).start()

    def store_desc(h, j):
      return pltpu.make_async_copy(
          outt_v.at[j % 2], out_hbm.at[h, :, pl.ds(b0, _IB)], osem)

    lanes = lax.iota(jnp.int32, _L)
    cvecs = [lanes + (g * _L) for g in range(d // _L)]

    for j in range(_NBUF):
      fire(j, j)

    def body(i, carry):
      for j in range(_NBUF):
        h = i * _NBUF + j
        pltpu.make_async_copy(
            table_hbm.at[idx_v.at[h]], rows_v.at[j], gsem).wait()
        rows2 = rows_v.at[j]
        outt2 = outt_v.at[j % 2]

        @pl.when(h >= 2)
        def _():
          store_desc(h, j).wait()

        @plsc.parallel_loop(0, _IB, step=1, unroll=8)
        def _(b):
          bsplat = jnp.full((_L,), b, jnp.int32)
          for g in range(d // _L):
            plsc.store_scatter(
                outt2, [cvecs[g], bsplat], rows2[b, pl.ds(g * _L, _L)])

        store_desc(h, j).start()

        @pl.when(h + _NBUF < hist)
        def _():
          fire(h + _NBUF, j)
      return carry

    lax.fori_loop(0, hist // _NBUF, body, 0)
    store_desc(0, 0).wait()
    store_desc(0, 1).wait()

  return lookup


def kernel(inputs, embeddings):
  b, h = inputs.shape
  vocab, d = embeddings.shape
  idx_t = jnp.transpose(inputs.astype(jnp.int32))
  tpad = jnp.pad(embeddings, ((0, 0), (0, _PW - d)))
  out = _build(h, b, vocab, d)(idx_t, tpad)
  return jnp.transpose(out, (2, 0, 1))
